# baseline (speedup 1.0000x reference)
import functools

import jax
import jax.numpy as jnp
from jax import lax
from jax.experimental import pallas as pl
from jax.experimental.pallas import tpu as pltpu
from jax.experimental.pallas import tpu_sc as plsc

_BATCH = 16384
_LANES = 16
_NUM_WORKERS = 32
_B_PER_W = _BATCH // _NUM_WORKERS
_IDX_ROWS = 4
_IDX_COLS = _B_PER_W // _IDX_ROWS
_CHUNKS = _B_PER_W // _LANES


def _sc_body(states_hbm, actions_hbm, table_hbm, out_hbm,
             states_v, actions_v, idx_v, vals_v, out_v, sem):
    wid = lax.axis_index("s") * 2 + lax.axis_index("c")
    base = wid * _B_PER_W

    pltpu.sync_copy(states_hbm.at[pl.ds(base, _B_PER_W)], states_v)
    pltpu.sync_copy(actions_hbm.at[pl.ds(base, _B_PER_W)], actions_v)

    for k in range(_CHUNKS):
        s = states_v[pl.ds(k * _LANES, _LANES)]
        a = actions_v[pl.ds(k * _LANES, _LANES)]
        e = s * 64 + a
        j = k // (_IDX_COLS // _LANES)
        off = (k % (_IDX_COLS // _LANES)) * _LANES
        idx_v[j, pl.ds(off, _LANES)] = e

    copies = [
        pltpu.make_async_copy(
            table_hbm.at[idx_v.at[j]],
            vals_v.at[pl.ds(j * _IDX_COLS, _IDX_COLS)],
            sem,
        )
        for j in range(_IDX_ROWS)
    ]
    for c in copies:
        c.start()
    for c in copies:
        c.wait()

    for k in range(_CHUNKS):
        x = vals_v[pl.ds(k * _LANES, _LANES)]
        out_v[pl.ds(k * _LANES, _LANES)] = 1.0 / (1.0 + jnp.exp(-x))

    pltpu.sync_copy(out_v, out_hbm.at[pl.ds(base, _B_PER_W)])


@functools.partial(jax.jit, static_argnames=())
def kernel(states, actions, rewards):
    table = rewards.reshape(-1)
    mesh = plsc.VectorSubcoreMesh(core_axis_name="c", subcore_axis_name="s")
    run = pl.kernel(
        _sc_body,
        mesh=mesh,
        out_type=jax.ShapeDtypeStruct((_BATCH,), jnp.float32),
        scratch_types=[
            pltpu.VMEM((_B_PER_W,), jnp.int32),
            pltpu.VMEM((_B_PER_W,), jnp.int32),
            pltpu.VMEM((_IDX_ROWS, _IDX_COLS), jnp.int32),
            pltpu.VMEM((_B_PER_W,), jnp.float32),
            pltpu.VMEM((_B_PER_W,), jnp.float32),
            pltpu.SemaphoreType.DMA,
        ],
    )
    return run(states, actions, table)


# speedup vs baseline: 10.7834x; 10.7834x over previous
import functools

import jax
import jax.numpy as jnp
from jax import lax
from jax.experimental import pallas as pl
from jax.experimental.pallas import tpu as pltpu
from jax.experimental.pallas import tpu_sc as plsc

_BATCH = 16384
_LANES = 16
_NUM_WORKERS = 32
_B_PER_W = _BATCH // _NUM_WORKERS
_CHUNK = 32
_N_CHUNKS = _B_PER_W // _CHUNK
_GROUPS = _CHUNK // _LANES


def _fire_chunk(tT_hbm, states_v, actions_v, buf, sem, c):
    for g in range(_GROUPS):
        i = c * _CHUNK + g * _LANES
        s = states_v[pl.ds(i, _LANES)]
        a = actions_v[pl.ds(i, _LANES)]
        ab = lax.shift_right_logical(a, 3) * 8
        col = lax.shift_right_logical(s, 7) * 128
        for j in range(_LANES):
            abj = pl.multiple_of(ab[j], 8)
            colj = pl.multiple_of(col[j], 128)
            row = (g * _LANES + j) * 8
            pltpu.make_async_copy(
                tT_hbm.at[pl.ds(abj, 8), pl.ds(colj, 128)],
                buf.at[pl.ds(row, 8)], sem,
            ).start()


def _drain_chunk(tT_hbm, buf, sem):
    for j in range(_CHUNK):
        pltpu.make_async_copy(
            tT_hbm.at[pl.ds(0, 8), pl.ds(0, 128)],
            buf.at[pl.ds(j * 8, 8)], sem,
        ).wait()


def _sc_body(states_hbm, actions_hbm, tT_hbm, out_hbm,
             states_v, actions_v, buf0, buf1, out_v, sem0, sem1):
    wid = lax.axis_index("s") * 2 + lax.axis_index("c")
    base = wid * _B_PER_W

    pltpu.sync_copy(states_hbm.at[pl.ds(base, _B_PER_W)], states_v)
    pltpu.sync_copy(actions_hbm.at[pl.ds(base, _B_PER_W)], actions_v)

    bufs = [buf0, buf1]
    sems = [sem0, sem1]
    eidx = lax.iota(jnp.int32, _LANES)
    _fire_chunk(tT_hbm, states_v, actions_v, bufs[0], sems[0], 0)
    for c in range(_N_CHUNKS):
        if c + 1 < _N_CHUNKS:
            _fire_chunk(tT_hbm, states_v, actions_v, bufs[(c + 1) % 2],
                        sems[(c + 1) % 2], c + 1)
        _drain_chunk(tT_hbm, bufs[c % 2], sems[c % 2])
        for g in range(_GROUPS):
            i = c * _CHUNK + g * _LANES
            s = states_v[pl.ds(i, _LANES)]
            a = actions_v[pl.ds(i, _LANES)]
            row = (g * _LANES + eidx) * 8 + lax.bitwise_and(a, 7)
            lane = lax.bitwise_and(s, 127)
            x = plsc.load_gather(bufs[c % 2], [row, lane])
            out_v[pl.ds(i, _LANES)] = 1.0 / (1.0 + jnp.exp(-x))

    pltpu.sync_copy(out_v, out_hbm.at[pl.ds(base, _B_PER_W)])


@functools.partial(jax.jit, static_argnames=())
def kernel(states, actions, rewards):
    tT = rewards.T
    mesh = plsc.VectorSubcoreMesh(core_axis_name="c", subcore_axis_name="s")
    run = pl.kernel(
        _sc_body,
        mesh=mesh,
        out_type=jax.ShapeDtypeStruct((_BATCH,), jnp.float32),
        scratch_types=[
            pltpu.VMEM((_B_PER_W,), jnp.int32),
            pltpu.VMEM((_B_PER_W,), jnp.int32),
            pltpu.VMEM((_CHUNK * 8, 128), jnp.float32),
            pltpu.VMEM((_CHUNK * 8, 128), jnp.float32),
            pltpu.VMEM((_B_PER_W,), jnp.float32),
            pltpu.SemaphoreType.DMA,
            pltpu.SemaphoreType.DMA,
        ],
        compiler_params=pltpu.CompilerParams(needs_layout_passes=False),
    )
    return run(states, actions, tT)


# speedup vs baseline: 12.4306x; 1.1528x over previous
import functools

import jax
import jax.numpy as jnp
from jax import lax
from jax.experimental import pallas as pl
from jax.experimental.pallas import tpu as pltpu
from jax.experimental.pallas import tpu_sc as plsc

_BATCH = 16384
_LANES = 16
_NUM_WORKERS = 32
_B_PER_W = _BATCH // _NUM_WORKERS
_CHUNK = 32
_N_CHUNKS = _B_PER_W // _CHUNK
_GROUPS = _CHUNK // _LANES


def _sc_body(states_hbm, actions_hbm, tT_hbm, out_hbm,
             states_v, actions_v, buf0, buf1, out_v, sem0, sem1):
    wid = lax.axis_index("s") * 2 + lax.axis_index("c")
    base = wid * _B_PER_W

    pltpu.sync_copy(states_hbm.at[pl.ds(base, _B_PER_W)], states_v)
    pltpu.sync_copy(actions_hbm.at[pl.ds(base, _B_PER_W)], actions_v)

    bufs = [buf0, buf1]
    sems = [sem0, sem1]
    eidx = lax.iota(jnp.int32, _LANES)

    def fire(ci, buf, sem):
        for g in range(_GROUPS):
            s = states_v[pl.ds(ci * _CHUNK + g * _LANES, _LANES)]
            a = actions_v[pl.ds(ci * _CHUNK + g * _LANES, _LANES)]
            ab = lax.shift_right_logical(a, 3) * 8
            col = lax.shift_right_logical(s, 7) * 128
            for j in range(_LANES):
                abj = pl.multiple_of(ab[j], 8)
                colj = pl.multiple_of(col[j], 128)
                row = (g * _LANES + j) * 8
                pltpu.make_async_copy(
                    tT_hbm.at[pl.ds(abj, 8), pl.ds(colj, 128)],
                    buf.at[pl.ds(row, 8)], sem,
                ).start()

    def drain(buf, sem):
        pltpu.make_async_copy(
            tT_hbm.at[pl.ds(0, _CHUNK * 8), pl.ds(0, 128)], buf, sem,
        ).wait()

    def extract(ci, buf):
        for g in range(_GROUPS):
            i = ci * _CHUNK + g * _LANES
            s = states_v[pl.ds(i, _LANES)]
            a = actions_v[pl.ds(i, _LANES)]
            row = (g * _LANES + eidx) * 8 + lax.bitwise_and(a, 7)
            lane = lax.bitwise_and(s, 127)
            x = plsc.load_gather(buf, [row, lane])
            out_v[pl.ds(i, _LANES)] = 1.0 / (1.0 + jnp.exp(-x))

    fire(0, bufs[0], sems[0])
    fire(1, bufs[1], sems[1])

    @pl.loop(0, _N_CHUNKS, step=2)
    def _chunks(c):
        for b in range(2):
            ci = c + b
            drain(bufs[b], sems[b])
            extract(ci, bufs[b])

            @pl.when(ci + 2 < _N_CHUNKS)
            def _refill():
                fire(ci + 2, bufs[b], sems[b])

    pltpu.sync_copy(out_v, out_hbm.at[pl.ds(base, _B_PER_W)])


@functools.partial(jax.jit, static_argnames=())
def kernel(states, actions, rewards):
    tT = rewards.T
    mesh = plsc.VectorSubcoreMesh(core_axis_name="c", subcore_axis_name="s")
    run = pl.kernel(
        _sc_body,
        mesh=mesh,
        out_type=jax.ShapeDtypeStruct((_BATCH,), jnp.float32),
        scratch_types=[
            pltpu.VMEM((_B_PER_W,), jnp.int32),
            pltpu.VMEM((_B_PER_W,), jnp.int32),
            pltpu.VMEM((_CHUNK * 8, 128), jnp.float32),
            pltpu.VMEM((_CHUNK * 8, 128), jnp.float32),
            pltpu.VMEM((_B_PER_W,), jnp.float32),
            pltpu.SemaphoreType.DMA,
            pltpu.SemaphoreType.DMA,
        ],
        compiler_params=pltpu.CompilerParams(needs_layout_passes=False),
    )
    return run(states, actions, tT)


# speedup vs baseline: 12.5374x; 1.0086x over previous
import functools

import jax
import jax.numpy as jnp
from jax import lax
from jax.experimental import pallas as pl
from jax.experimental.pallas import tpu as pltpu
from jax.experimental.pallas import tpu_sc as plsc

_BATCH = 16384
_LANES = 16
_NUM_WORKERS = 32
_B_PER_W = _BATCH // _NUM_WORKERS
_CHUNK = 32
_N_CHUNKS = _B_PER_W // _CHUNK
_GROUPS = _CHUNK // _LANES


def _sc_body(states_hbm, actions_hbm, tT_hbm, out_hbm,
             states_v, actions_v, buf0, buf1, out_v, sem0, sem1):
    wid = lax.axis_index("s") * 2 + lax.axis_index("c")
    base = wid * _B_PER_W

    pltpu.sync_copy(states_hbm.at[pl.ds(base, _B_PER_W)], states_v)
    pltpu.sync_copy(actions_hbm.at[pl.ds(base, _B_PER_W)], actions_v)

    bufs = [buf0, buf1]
    sems = [sem0, sem1]
    eidx = lax.iota(jnp.int32, _LANES)

    def fire(ci, buf, sem):
        for g in range(_GROUPS):
            s = states_v[pl.ds(ci * _CHUNK + g * _LANES, _LANES)]
            a = actions_v[pl.ds(ci * _CHUNK + g * _LANES, _LANES)]
            ab = lax.shift_right_logical(a, 3) * 8
            col = lax.shift_right_logical(s, 7) * 128
            for j in range(_LANES):
                abj = pl.multiple_of(ab[j], 8)
                colj = pl.multiple_of(col[j], 128)
                row = (g * _LANES + j) * 8
                pltpu.make_async_copy(
                    tT_hbm.at[pl.ds(abj, 8), pl.ds(colj, 128)],
                    buf.at[pl.ds(row, 8)], sem,
                ).start()

    def drain(buf, sem):
        pltpu.make_async_copy(
            tT_hbm.at[pl.ds(0, _CHUNK * 8), pl.ds(0, 128)], buf, sem,
        ).wait()

    def extract(ci, buf):
        for g in range(_GROUPS):
            i = ci * _CHUNK + g * _LANES
            s = states_v[pl.ds(i, _LANES)]
            a = actions_v[pl.ds(i, _LANES)]
            row = (g * _LANES + eidx) * 8 + lax.bitwise_and(a, 7)
            lane = lax.bitwise_and(s, 127)
            x = plsc.load_gather(buf, [row, lane])
            out_v[pl.ds(i, _LANES)] = 1.0 / (1.0 + jnp.exp(-x))

    @pl.loop(-2, _N_CHUNKS, step=2)
    def _chunks(c):
        for b in range(2):
            ci = c + b

            @pl.when(ci >= 0)
            def _consume():
                drain(bufs[b], sems[b])
                extract(ci, bufs[b])

            @pl.when(ci + 2 < _N_CHUNKS)
            def _refill():
                fire(ci + 2, bufs[b], sems[b])

    pltpu.sync_copy(out_v, out_hbm.at[pl.ds(base, _B_PER_W)])


@functools.partial(jax.jit, static_argnames=())
def kernel(states, actions, rewards):
    tT = rewards.T
    mesh = plsc.VectorSubcoreMesh(core_axis_name="c", subcore_axis_name="s")
    run = pl.kernel(
        _sc_body,
        mesh=mesh,
        out_type=jax.ShapeDtypeStruct((_BATCH,), jnp.float32),
        scratch_types=[
            pltpu.VMEM((_B_PER_W,), jnp.int32),
            pltpu.VMEM((_B_PER_W,), jnp.int32),
            pltpu.VMEM((_CHUNK * 8, 128), jnp.float32),
            pltpu.VMEM((_CHUNK * 8, 128), jnp.float32),
            pltpu.VMEM((_B_PER_W,), jnp.float32),
            pltpu.SemaphoreType.DMA,
            pltpu.SemaphoreType.DMA,
        ],
        compiler_params=pltpu.CompilerParams(needs_layout_passes=False),
    )
    return run(states, actions, tT)
